# Initial kernel scaffold; baseline (speedup 1.0000x reference)
import jax
import jax.numpy as jnp
from jax.experimental import pallas as pl


def kernel(fatoms, fbonds, W_ein, W_edge, W_eout, out_n, bgraph, aingraph):
    raise NotImplementedError("write your pallas kernel here")


# speedup vs baseline: 4.1277x; 4.1277x over previous
import functools

import jax
import jax.numpy as jnp
from jax import lax
from jax.experimental import pallas as pl
from jax.experimental.pallas import tpu as pltpu
from jax.experimental.pallas import tpu_sc as plsc

ATOM_FDIM = 39
N_ATOMS = 10000
N_BONDS = 160000
MAX_NB = 6
HIDDEN = 256
OUT = 256
DEPTH = 3

_AF_PAD = 128
_AT_ROWS = N_ATOMS + 8

_NW = 32
_C = 40


def _sc_info():
    info = plsc.get_sparse_core_info()
    return info.num_cores, info.num_subcores


def _make_gather6(T, D, OUTR, reduce_sum):
    nc, ns = _sc_info()
    nw = nc * ns
    per_w = OUTR // nw
    if reduce_sum:
        steps = per_w // _C
    else:
        steps = per_w // (_C * MAX_NB)
    vpr = D // 16

    mesh = plsc.VectorSubcoreMesh(core_axis_name="c", subcore_axis_name="s")

    scratch = [
        pltpu.VMEM((MAX_NB, _C), jnp.int32),
        pltpu.VMEM((MAX_NB * _C, D), jnp.float32),
        pltpu.SemaphoreType.DMA,
    ]
    if reduce_sum:
        scratch.insert(2, pltpu.VMEM((_C, D), jnp.float32))

    def body_reduce(table, idxr, out, idx_v, gbuf, acc, sem):
        wid = lax.axis_index("s") * nc + lax.axis_index("c")
        base = wid * per_w

        def step(s, carry):
            pltpu.sync_copy(idxr.at[wid, s], idx_v)
            cps = [
                pltpu.async_copy(table.at[idx_v.at[k]],
                                 gbuf.at[pl.ds(k * _C, _C)], sem)
                for k in range(MAX_NB)
            ]
            for cp in cps:
                cp.wait()

            def row(i, c2):
                for v in range(vpr):
                    sl = pl.ds(v * 16, 16)
                    x = gbuf[i, sl]
                    for k in range(1, MAX_NB):
                        x = x + gbuf[k * _C + i, sl]
                    acc[i, sl] = x
                return c2

            lax.fori_loop(0, _C, row, 0)
            pltpu.sync_copy(acc, out.at[pl.ds(base + s * _C, _C)])
            return carry

        lax.fori_loop(0, steps, step, 0)

    def body_copy(table, idxr, out, idx_v, gbuf, sem):
        wid = lax.axis_index("s") * nc + lax.axis_index("c")
        base = wid * per_w

        def step(s, carry):
            pltpu.sync_copy(idxr.at[wid, s], idx_v)
            cps = [
                pltpu.async_copy(table.at[idx_v.at[k]],
                                 gbuf.at[pl.ds(k * _C, _C)], sem)
                for k in range(MAX_NB)
            ]
            for cp in cps:
                cp.wait()
            pltpu.sync_copy(
                gbuf, out.at[pl.ds(base + s * _C * MAX_NB, _C * MAX_NB)])
            return carry

        lax.fori_loop(0, steps, step, 0)

    body = body_reduce if reduce_sum else body_copy
    return pl.kernel(
        body,
        out_type=jax.ShapeDtypeStruct((OUTR, D), jnp.float32),
        mesh=mesh,
        scratch_types=scratch,
    )


def _pack_idx6(idx2d, outr):
    r = idx2d.shape[0]
    if r < outr:
        idx2d = jnp.pad(idx2d, ((0, outr - r), (0, 0)))
    per_w = outr // _NW
    s = per_w // _C
    return idx2d.reshape(_NW, s, _C, MAX_NB).transpose(0, 1, 3, 2)


def _pack_idx1(idx1d, outr):
    r = idx1d.shape[0]
    if r < outr:
        idx1d = jnp.pad(idx1d, (0, outr - r))
    per_w = outr // _NW
    s = per_w // (_C * MAX_NB)
    return idx1d.reshape(_NW, s, MAX_NB, _C)


_TC_R = 1600


def _h0_body(fb_ref, w_ref, o_ref):
    o_ref[...] = jax.nn.relu(
        jnp.dot(fb_ref[...], w_ref[...], preferred_element_type=jnp.float32))


def _q0_body(h0_ref, am_ref, wh_ref, wa_ref, o_ref):
    o_ref[...] = (
        jnp.dot(h0_ref[...], wh_ref[...], preferred_element_type=jnp.float32)
        + jnp.dot(am_ref[...], wa_ref[...], preferred_element_type=jnp.float32))


def _q_body(h0_ref, s_ref, am_ref, wh_ref, wa_ref, o_ref):
    h = jax.nn.relu(h0_ref[...] + s_ref[...])
    o_ref[...] = (
        jnp.dot(h, wh_ref[...], preferred_element_type=jnp.float32)
        + jnp.dot(am_ref[...], wa_ref[...], preferred_element_type=jnp.float32))


def _h3_body(h0_ref, s_ref, o_ref):
    o_ref[...] = jax.nn.relu(h0_ref[...] + s_ref[...])


def _out_body(sh_ref, sa_ref, vh_ref, va_ref, o_ref):
    dn = (((1,), (1,)), ((), ()))
    o_ref[...] = jax.nn.relu(
        lax.dot_general(vh_ref[...], sh_ref[...], dn,
                        preferred_element_type=jnp.float32)
        + lax.dot_general(va_ref[...], sa_ref[...], dn,
                          preferred_element_type=jnp.float32))


def _row_spec(d):
    return pl.BlockSpec((_TC_R, d), lambda i: (i, 0))


def _full_spec(shape):
    return pl.BlockSpec(shape, lambda i: (0, 0))


def kernel(fatoms, fbonds, W_ein, W_edge, W_eout, out_n, bgraph, aingraph):
    E, N = N_BONDS, N_ATOMS
    grid = (E // _TC_R,)
    params = pltpu.CompilerParams(dimension_semantics=("parallel",))

    fb16 = jnp.pad(fbonds, ((0, 0), (0, 16 - fbonds.shape[1])))
    wein_t = jnp.pad(W_ein.T, ((0, 16 - W_ein.shape[1]), (0, 0)))
    W_h = W_edge[:, :HIDDEN]
    W_a = W_edge[:, HIDDEN:]
    V_h = W_eout[:, :HIDDEN]
    V_a = W_eout[:, HIDDEN:]
    wa_t = jnp.pad(W_a.T, ((0, _AF_PAD - ATOM_FDIM), (0, 0)))
    va64 = jnp.pad(V_a, ((0, 0), (0, _AF_PAD - ATOM_FDIM)))
    fat64 = jnp.pad(
        fatoms, ((0, _AT_ROWS - N), (0, _AF_PAD - ATOM_FDIM)))
    idx_full = jnp.concatenate(
        [jnp.full((1,), N, jnp.int32), out_n.astype(jnp.int32)])

    am_rows = 161280
    idx_am = _pack_idx1(idx_full, am_rows)
    idx_bg = _pack_idx6(bgraph.astype(jnp.int32), E)
    ain_rows = 10240
    idx_ain = _pack_idx6(aingraph.astype(jnp.int32), ain_rows)

    h0 = pl.pallas_call(
        _h0_body,
        grid=grid,
        in_specs=[_row_spec(16), _full_spec((16, HIDDEN))],
        out_specs=_row_spec(HIDDEN),
        out_shape=jax.ShapeDtypeStruct((E, HIDDEN), jnp.float32),
        compiler_params=params,
    )(fb16, wein_t)

    amraw_pad = _make_gather6(_AT_ROWS, _AF_PAD, am_rows, reduce_sum=False)(
        fat64, idx_am)
    amraw = amraw_pad[:E]

    g6_bond = _make_gather6(E, HIDDEN, E, reduce_sum=True)
    q_call = pl.pallas_call(
        _q_body,
        grid=grid,
        in_specs=[_row_spec(HIDDEN), _row_spec(HIDDEN), _row_spec(_AF_PAD),
                  _full_spec((HIDDEN, HIDDEN)), _full_spec((_AF_PAD, HIDDEN))],
        out_specs=_row_spec(HIDDEN),
        out_shape=jax.ShapeDtypeStruct((E, HIDDEN), jnp.float32),
        compiler_params=params,
    )
    q = pl.pallas_call(
        _q0_body,
        grid=grid,
        in_specs=[_row_spec(HIDDEN), _row_spec(_AF_PAD),
                  _full_spec((HIDDEN, HIDDEN)), _full_spec((_AF_PAD, HIDDEN))],
        out_specs=_row_spec(HIDDEN),
        out_shape=jax.ShapeDtypeStruct((E, HIDDEN), jnp.float32),
        compiler_params=params,
    )(h0, amraw, W_h.T, wa_t)
    for _ in range(DEPTH - 1):
        s = g6_bond(q, idx_bg)
        q = q_call(h0, s, amraw, W_h.T, wa_t)
    s = g6_bond(q, idx_bg)

    h3 = pl.pallas_call(
        _h3_body,
        grid=grid,
        in_specs=[_row_spec(HIDDEN), _row_spec(HIDDEN)],
        out_specs=_row_spec(HIDDEN),
        out_shape=jax.ShapeDtypeStruct((E, HIDDEN), jnp.float32),
        compiler_params=params,
    )(h0, s)

    s_h = _make_gather6(E, HIDDEN, ain_rows, reduce_sum=True)(
        h3, idx_ain)[:N]
    s_a = _make_gather6(am_rows, _AF_PAD, ain_rows, reduce_sum=True)(
        amraw_pad, idx_ain)[:N]

    out = pl.pallas_call(
        _out_body,
        grid=(1,),
        in_specs=[_full_spec((N, HIDDEN)), _full_spec((N, _AF_PAD)),
                  _full_spec((OUT, HIDDEN)), _full_spec((OUT, _AF_PAD))],
        out_specs=_full_spec((OUT, N)),
        out_shape=jax.ShapeDtypeStruct((OUT, N), jnp.float32),
        compiler_params=params,
    )(s_h, s_a, V_h, va64)
    return out
